# baseline (speedup 1.0000x reference)
import jax
import jax.numpy as jnp
from jax.experimental import pallas as pl
from jax.experimental.pallas import tpu as pltpu

_TOP_K = 8
_ROWS_PER_BLOCK = 512


def _router_block(hs_ref, gw_ref, w_ref, i_ref, logits_ref):
    hs = hs_ref[...]
    gw = gw_ref[...]
    logits = jax.lax.dot_general(
        hs, gw, (((1,), (1,)), ((), ())),
        preferred_element_type=jnp.float32,
    )
    logits_ref[...] = logits

    rows, n_exp = logits.shape
    idxs = jax.lax.broadcasted_iota(jnp.int32, (rows, n_exp), 1)
    x = logits
    vals = []
    inds = []
    for _ in range(_TOP_K):
        m = jnp.max(x, axis=1, keepdims=True)
        cand = jnp.where(x == m, idxs, n_exp)
        am = jnp.min(cand, axis=1, keepdims=True)
        vals.append(m)
        inds.append(am)
        x = jnp.where(idxs == am, -jnp.inf, x)
    v = jnp.concatenate(vals, axis=1)
    e = jnp.exp(v - v[:, :1])
    w_ref[...] = e / jnp.sum(e, axis=1, keepdims=True)
    i_ref[...] = jnp.concatenate(inds, axis=1)


@jax.jit
def kernel(hidden_states, gate_weight):
    tokens, dim = hidden_states.shape
    n_exp = gate_weight.shape[0]
    r = _ROWS_PER_BLOCK
    w, i, logits = pl.pallas_call(
        _router_block,
        grid=(tokens // r,),
        in_specs=[
            pl.BlockSpec((r, dim), lambda b: (b, 0)),
            pl.BlockSpec((n_exp, dim), lambda b: (0, 0)),
        ],
        out_specs=(
            pl.BlockSpec((r, _TOP_K), lambda b: (b, 0)),
            pl.BlockSpec((r, _TOP_K), lambda b: (b, 0)),
            pl.BlockSpec((r, n_exp), lambda b: (b, 0)),
        ),
        out_shape=(
            jax.ShapeDtypeStruct((tokens, _TOP_K), jnp.float32),
            jax.ShapeDtypeStruct((tokens, _TOP_K), jnp.int32),
            jax.ShapeDtypeStruct((tokens, n_exp), jnp.float32),
        ),
        compiler_params=pltpu.CompilerParams(
            dimension_semantics=("arbitrary",),
        ),
    )(hidden_states, gate_weight)
    return (w, i, logits)
